# Initial kernel scaffold; baseline (speedup 1.0000x reference)
import jax
import jax.numpy as jnp
from jax.experimental import pallas as pl


def kernel(x, edge_index, W, b):
    raise NotImplementedError("write your pallas kernel here")


# speedup vs baseline: 4.2300x; 4.2300x over previous
import functools

import jax
import jax.numpy as jnp
from jax import lax
from jax.experimental import pallas as pl
from jax.experimental.pallas import tpu as pltpu
from jax.experimental.pallas import tpu_sc as plsc

_NC = 2
_NS = 16
_NW = _NC * _NS
_L = 16
_C = 128


def _scatter_call(x, src, dst, zeros):
    n, d = x.shape
    n_pad = zeros.shape[0]
    e = src.shape[0]
    n_chunks = e // _C
    iters = (n_chunks + _NW - 1) // _NW
    rows_per_tile = n_pad // _NS
    mesh = plsc.VectorSubcoreMesh(core_axis_name="c", subcore_axis_name="s")

    @functools.partial(
        pl.kernel,
        out_type=jax.ShapeDtypeStruct((_NC, n_pad, d), jnp.float32),
        mesh=mesh,
        scratch_types=[
            pltpu.VMEM((_C,), jnp.int32),
            pltpu.VMEM((1, _C), jnp.int32),
            pltpu.VMEM((_C, d), jnp.float32),
            pltpu.VMEM_SHARED((n_pad, d), jnp.float32),
            pltpu.SemaphoreType.DMA,
        ],
    )
    def scatter_k(x_hbm, src_hbm, dst_hbm, zeros_hbm, s_out_hbm,
                  src_v, dst_v, rows_v, s_sh, sem):
        core = lax.axis_index("c")
        sub = lax.axis_index("s")
        wid = sub * _NC + core
        r0 = sub * rows_per_tile
        pltpu.sync_copy(zeros_hbm.at[pl.ds(r0, rows_per_tile)],
                        s_sh.at[pl.ds(r0, rows_per_tile)])
        plsc.subcore_barrier()

        def body(i, carry):
            cidx = wid + i * _NW

            @pl.when(cidx < n_chunks)
            def _():
                base = cidx * _C
                pltpu.sync_copy(src_hbm.at[pl.ds(base, _C)], src_v)
                pltpu.sync_copy(dst_hbm.at[pl.ds(base, _C)], dst_v.at[0])
                pltpu.async_copy(x_hbm.at[src_v], rows_v, sem).wait()
                pltpu.sync_copy(rows_v, s_sh.at[dst_v.at[0]], add=True)

            return carry

        lax.fori_loop(0, iters, body, 0)
        plsc.subcore_barrier()
        pltpu.sync_copy(s_sh.at[pl.ds(r0, rows_per_tile)],
                        s_out_hbm.at[core, pl.ds(r0, rows_per_tile)])

    return scatter_k(x, src, dst, zeros)


def _matmul_call(x, s_parts, w, b):
    n, d = x.shape
    dout = w.shape[1]
    bm = 1000

    def mm_body(x_ref, s_ref, w_ref, b_ref, y1_ref, y2_ref):
        wmat = w_ref[...]
        y1_ref[...] = (
            jnp.dot(x_ref[...], wmat[:d, :], preferred_element_type=jnp.float32)
            + b_ref[...]
        )
        s_sum = s_ref[0] + s_ref[1]
        y2_ref[...] = jnp.dot(s_sum, wmat[d:, :],
                              preferred_element_type=jnp.float32)

    return pl.pallas_call(
        mm_body,
        grid=(n // bm,),
        in_specs=[
            pl.BlockSpec((bm, d), lambda i: (i, 0)),
            pl.BlockSpec((_NC, bm, d), lambda i: (0, i, 0)),
            pl.BlockSpec((2 * d, dout), lambda i: (0, 0)),
            pl.BlockSpec((1, dout), lambda i: (0, 0)),
        ],
        out_specs=[
            pl.BlockSpec((bm, dout), lambda i: (i, 0)),
            pl.BlockSpec((bm, dout), lambda i: (i, 0)),
        ],
        out_shape=[
            jax.ShapeDtypeStruct((n, dout), jnp.float32),
            jax.ShapeDtypeStruct((n, dout), jnp.float32),
        ],
    )(x, s_parts, w, b.reshape(1, dout))


def _gather_call(y1, y2, src, dst):
    n, d = y1.shape
    e = src.shape[0]
    n_chunks = e // _C
    iters = (n_chunks + _NW - 1) // _NW
    mesh = plsc.VectorSubcoreMesh(core_axis_name="c", subcore_axis_name="s")

    @functools.partial(
        pl.kernel,
        out_type=jax.ShapeDtypeStruct((e, d), jnp.float32),
        mesh=mesh,
        scratch_types=[
            pltpu.VMEM((_C,), jnp.int32),
            pltpu.VMEM((_C,), jnp.int32),
            pltpu.VMEM((_C, d), jnp.float32),
            pltpu.VMEM((_C, d), jnp.float32),
            pltpu.SemaphoreType.DMA,
            pltpu.SemaphoreType.DMA,
        ],
    )
    def gather_k(y1_hbm, y2_hbm, src_hbm, dst_hbm, out_hbm,
                 src_v, dst_v, a_v, b_v, sem1, sem2):
        core = lax.axis_index("c")
        sub = lax.axis_index("s")
        wid = sub * _NC + core

        def body(i, carry):
            cidx = wid + i * _NW

            @pl.when(cidx < n_chunks)
            def _():
                base = cidx * _C
                pltpu.sync_copy(src_hbm.at[pl.ds(base, _C)], src_v)
                pltpu.sync_copy(dst_hbm.at[pl.ds(base, _C)], dst_v)
                cp1 = pltpu.async_copy(y1_hbm.at[src_v], a_v, sem1)
                cp2 = pltpu.async_copy(y2_hbm.at[dst_v], b_v, sem2)
                cp1.wait()
                cp2.wait()

                def add_row(r, c2):
                    for j in range(d // _L):
                        sl = pl.ds(j * _L, _L)
                        a_v[r, sl] = a_v[r, sl] + b_v[r, sl]
                    return c2

                lax.fori_loop(0, _C, add_row, 0)
                pltpu.sync_copy(a_v, out_hbm.at[pl.ds(base, _C)])

            return carry

        lax.fori_loop(0, iters, body, 0)

    return gather_k(y1, y2, src, dst)


def kernel(x, edge_index, W, b):
    src = edge_index[0].astype(jnp.int32)
    dst = edge_index[1].astype(jnp.int32)
    n, d = x.shape
    n_pad = ((n + 8 * _NS - 1) // (8 * _NS)) * (8 * _NS)
    zeros = jnp.zeros((n_pad, d), jnp.float32)
    s_parts = _scatter_call(x, src, dst, zeros)
    y1, y2 = _matmul_call(x, s_parts, W, b)
    return _gather_call(y1, y2, src, dst)
